# baseline (speedup 1.0000x reference)
import functools

import jax
import jax.numpy as jnp
from jax import lax
from jax.experimental import pallas as pl
from jax.experimental.pallas import tpu as pltpu
from jax.experimental.pallas import tpu_sc as plsc

N = 10000
E = 320000
D = 128
H = D // 2

NC = 2
NS = 16
NW = NC * NS
CH = 125
EPT_D = E // NW
NCHUNK_D = EPT_D // CH
EPT_S = E // NS
NCHUNK_S = EPT_S // CH
NBUF = 5
RPS = N // NS
ZR = 125
DEGW = 16

_mesh = plsc.VectorSubcoreMesh(core_axis_name="c", subcore_axis_name="s")
_sc_params = pltpu.CompilerParams(use_tc_tiling_on_sc=False)


@functools.partial(
    pl.kernel,
    out_type=jax.ShapeDtypeStruct((NC, NS, RPS, DEGW), jnp.float32),
    mesh=_mesh,
    scratch_types=[
        pltpu.VMEM((NCHUNK_D, CH), jnp.int32),
        pltpu.VMEM((CH, DEGW), jnp.float32),
        pltpu.VMEM((RPS, DEGW), jnp.float32),
        pltpu.VMEM_SHARED((N, DEGW), jnp.float32),
    ],
    compiler_params=_sc_params,
)
def _deg_kernel(dst_hbm, out_hbm, idx_v, ones_v, z_v, acc):
    cid = lax.axis_index("c")
    sid = lax.axis_index("s")
    wid = cid * NS + sid

    @pl.loop(0, RPS)
    def _(i):
        z_v[i, :] = jnp.zeros((DEGW,), jnp.float32)

    pltpu.sync_copy(z_v, acc.at[pl.ds(sid * RPS, RPS)])

    @pl.loop(0, CH)
    def _(i):
        ones_v[i, :] = jnp.ones((DEGW,), jnp.float32)

    pltpu.sync_copy(dst_hbm.at[wid], idx_v)
    plsc.subcore_barrier()

    @pl.loop(0, NCHUNK_D)
    def _(g):
        pltpu.sync_copy(ones_v, acc.at[idx_v.at[g]], add=True)

    plsc.subcore_barrier()
    pltpu.sync_copy(acc.at[pl.ds(sid * RPS, RPS)], out_hbm.at[cid].at[sid])


@functools.partial(
    pl.kernel,
    out_type=jax.ShapeDtypeStruct((NC, NS, RPS, H), jnp.float32),
    mesh=_mesh,
    scratch_types=[
        pltpu.VMEM((NCHUNK_S, CH), jnp.int32),
        pltpu.VMEM((NCHUNK_S, CH), jnp.int32),
        [pltpu.VMEM((CH, H), jnp.float32)] * NBUF,
        [pltpu.SemaphoreType.DMA] * NBUF,
        pltpu.VMEM_SHARED((N, H), jnp.float32),
    ],
    compiler_params=_sc_params,
)
def _seg_kernel(y_hbm, src_hbm, dst_hbm, out_hbm,
                src_v, dst_v, bufs, sems, acc):
    cid = lax.axis_index("c")
    sid = lax.axis_index("s")

    @pl.loop(0, CH)
    def _(i):
        @pl.loop(0, H, step=16)
        def _(j):
            bufs[0][i, pl.ds(j, 16)] = jnp.zeros((16,), jnp.float32)

    @pl.loop(0, RPS, step=CH)
    def _(k):
        pltpu.sync_copy(bufs[0], acc.at[pl.ds(sid * RPS + k, CH)])

    pltpu.sync_copy(src_hbm.at[sid], src_v)
    pltpu.sync_copy(dst_hbm.at[sid], dst_v)
    plsc.subcore_barrier()

    y_half = y_hbm.at[cid]

    for k in range(NBUF - 1):
        pltpu.async_copy(y_half.at[src_v.at[k]], bufs[k], sems[k])

    @pl.loop(0, NCHUNK_S // NBUF)
    def _(i):
        g = NBUF * i
        for k in range(NBUF):
            buf, sem = bufs[k], sems[k]
            nbuf = bufs[(k + NBUF - 1) % NBUF]
            nsem = sems[(k + NBUF - 1) % NBUF]
            pltpu.make_async_copy(y_half.at[src_v.at[g + k]], buf,
                                  sem).wait()
            if k == 0:
                pltpu.async_copy(y_half.at[src_v.at[g + NBUF - 1]], nbuf,
                                 nsem)
            else:
                @pl.when(g + k + NBUF - 1 < NCHUNK_S)
                def _():
                    pltpu.async_copy(y_half.at[src_v.at[g + k + NBUF - 1]],
                                     nbuf, nsem)
            pltpu.sync_copy(buf, acc.at[dst_v.at[g + k]], add=True)

    plsc.subcore_barrier()
    pltpu.sync_copy(acc.at[pl.ds(sid * RPS, RPS)], out_hbm.at[cid].at[sid])


RB = 1000


def _prep_body(x_ref, w_ref, d0_ref, d1_ref, y_ref, dis_ref):
    dis = lax.rsqrt(1.0 + d0_ref[...] + d1_ref[...])
    dis_ref[...] = dis
    xw = jnp.dot(x_ref[...], w_ref[...], preferred_element_type=jnp.float32)
    y_ref[0, ...] = dis * xw[:, :H]
    y_ref[1, ...] = dis * xw[:, H:]


def _prep(x, w, d0, d1):
    return pl.pallas_call(
        _prep_body,
        grid=(N // RB,),
        in_specs=[pl.BlockSpec((RB, D), lambda i: (i, 0)),
                  pl.BlockSpec((D, D), lambda i: (0, 0)),
                  pl.BlockSpec((RB, 1), lambda i: (i, 0)),
                  pl.BlockSpec((RB, 1), lambda i: (i, 0))],
        out_specs=[pl.BlockSpec((NC, RB, H), lambda i: (0, i, 0)),
                   pl.BlockSpec((RB, 1), lambda i: (i, 0))],
        out_shape=[jax.ShapeDtypeStruct((NC, N, H), jnp.float32),
                   jax.ShapeDtypeStruct((N, 1), jnp.float32)],
    )(x, w, d0, d1)


def _mid_body(s_ref, y_ref, dis_ref, b_ref, w_ref, y2_ref):
    dis = dis_ref[...]
    h0 = jnp.maximum(dis * (s_ref[0, ...] + y_ref[0, ...]) + b_ref[:, :H], 0.0)
    h1 = jnp.maximum(dis * (s_ref[1, ...] + y_ref[1, ...]) + b_ref[:, H:], 0.0)
    xw = (jnp.dot(h0, w_ref[:H, :], preferred_element_type=jnp.float32)
          + jnp.dot(h1, w_ref[H:, :], preferred_element_type=jnp.float32))
    y2_ref[0, ...] = dis * xw[:, :H]
    y2_ref[1, ...] = dis * xw[:, H:]


def _mid(s, y, dis, b, w):
    return pl.pallas_call(
        _mid_body,
        grid=(N // RB,),
        in_specs=[pl.BlockSpec((NC, RB, H), lambda i: (0, i, 0)),
                  pl.BlockSpec((NC, RB, H), lambda i: (0, i, 0)),
                  pl.BlockSpec((RB, 1), lambda i: (i, 0)),
                  pl.BlockSpec((1, D), lambda i: (0, 0)),
                  pl.BlockSpec((D, D), lambda i: (0, 0))],
        out_specs=pl.BlockSpec((NC, RB, H), lambda i: (0, i, 0)),
        out_shape=jax.ShapeDtypeStruct((NC, N, H), jnp.float32),
    )(s, y, dis, b, w)


def _fin_body(s_ref, y_ref, dis_ref, b_ref, o_ref):
    dis = dis_ref[...]
    o_ref[:, :H] = dis * (s_ref[0, ...] + y_ref[0, ...]) + b_ref[:, :H]
    o_ref[:, H:] = dis * (s_ref[1, ...] + y_ref[1, ...]) + b_ref[:, H:]


def _fin(s, y, dis, b):
    return pl.pallas_call(
        _fin_body,
        grid=(N // RB,),
        in_specs=[pl.BlockSpec((NC, RB, H), lambda i: (0, i, 0)),
                  pl.BlockSpec((NC, RB, H), lambda i: (0, i, 0)),
                  pl.BlockSpec((RB, 1), lambda i: (i, 0)),
                  pl.BlockSpec((1, D), lambda i: (0, 0))],
        out_specs=pl.BlockSpec((RB, D), lambda i: (i, 0)),
        out_shape=jax.ShapeDtypeStruct((N, D), jnp.float32),
    )(s, y, dis, b)


def kernel(x, edge_idx, W1, b1, W2, b2):
    src3 = edge_idx[0].reshape(NS, NCHUNK_S, CH)
    dst3 = edge_idx[1].reshape(NS, NCHUNK_S, CH)
    dst3d = edge_idx[1].reshape(NW, NCHUNK_D, CH)

    deg_parts = _deg_kernel(dst3d)
    deg_parts = deg_parts.reshape(NC, N, DEGW)
    d0 = deg_parts[0, :, 0:1]
    d1 = deg_parts[1, :, 0:1]

    y1, dis = _prep(x, W1, d0, d1)

    s1 = _seg_kernel(y1, src3, dst3).reshape(NC, N, H)
    y2 = _mid(s1, y1, dis, b1.reshape(1, D), W2)

    s2 = _seg_kernel(y2, src3, dst3).reshape(NC, N, H)
    return _fin(s2, y2, dis, b2.reshape(1, D))


# speedup vs baseline: 1.0301x; 1.0301x over previous
import functools

import jax
import jax.numpy as jnp
from jax import lax
from jax.experimental import pallas as pl
from jax.experimental.pallas import tpu as pltpu
from jax.experimental.pallas import tpu_sc as plsc

N = 10000
E = 320000
D = 128
H = D // 2

NC = 2
NS = 16
NW = NC * NS
CH = 125
EPT_D = E // NW
NCHUNK_D = EPT_D // CH
EPT_S = E // NS
NCHUNK_S = EPT_S // CH
NBUF = 5
RPS = N // NS
ZR = 125
DEGW = 16

_mesh = plsc.VectorSubcoreMesh(core_axis_name="c", subcore_axis_name="s")
_sc_params = pltpu.CompilerParams(use_tc_tiling_on_sc=False)


@functools.partial(
    pl.kernel,
    out_type=jax.ShapeDtypeStruct((NC, NS, RPS, DEGW), jnp.float32),
    mesh=_mesh,
    scratch_types=[
        pltpu.VMEM((NCHUNK_D, CH), jnp.int32),
        pltpu.VMEM((CH, DEGW), jnp.float32),
        pltpu.VMEM((RPS, DEGW), jnp.float32),
        pltpu.VMEM_SHARED((N, DEGW), jnp.float32),
        pltpu.SemaphoreType.DMA,
    ],
    compiler_params=_sc_params,
)
def _deg_kernel(dst_hbm, out_hbm, idx_v, ones_v, z_v, acc, sem):
    cid = lax.axis_index("c")
    sid = lax.axis_index("s")
    wid = cid * NS + sid

    @pl.loop(0, RPS)
    def _(i):
        z_v[i, :] = jnp.zeros((DEGW,), jnp.float32)

    pltpu.sync_copy(z_v, acc.at[pl.ds(sid * RPS, RPS)])

    @pl.loop(0, CH)
    def _(i):
        ones_v[i, :] = jnp.ones((DEGW,), jnp.float32)

    pltpu.sync_copy(dst_hbm.at[wid], idx_v)
    plsc.subcore_barrier()

    @pl.loop(0, NCHUNK_D)
    def _(g):
        pltpu.async_copy(ones_v, acc.at[idx_v.at[g]], sem, add=True)

    @pl.loop(0, NCHUNK_D)
    def _(g):
        pltpu.make_async_copy(ones_v, acc.at[idx_v.at[g]], sem).wait()

    plsc.subcore_barrier()
    pltpu.sync_copy(acc.at[pl.ds(sid * RPS, RPS)], out_hbm.at[cid].at[sid])


@functools.partial(
    pl.kernel,
    out_type=jax.ShapeDtypeStruct((NC, NS, RPS, H), jnp.float32),
    mesh=_mesh,
    scratch_types=[
        pltpu.VMEM((NCHUNK_S, CH), jnp.int32),
        pltpu.VMEM((NCHUNK_S, CH), jnp.int32),
        [pltpu.VMEM((CH, H), jnp.float32)] * NBUF,
        [pltpu.SemaphoreType.DMA] * NBUF,
        pltpu.VMEM_SHARED((N, H), jnp.float32),
    ],
    compiler_params=_sc_params,
)
def _seg_kernel(y_hbm, src_hbm, dst_hbm, out_hbm,
                src_v, dst_v, bufs, sems, acc):
    cid = lax.axis_index("c")
    sid = lax.axis_index("s")

    @pl.loop(0, CH)
    def _(i):
        @pl.loop(0, H, step=16)
        def _(j):
            bufs[0][i, pl.ds(j, 16)] = jnp.zeros((16,), jnp.float32)

    @pl.loop(0, RPS, step=CH)
    def _(k):
        pltpu.sync_copy(bufs[0], acc.at[pl.ds(sid * RPS + k, CH)])

    pltpu.sync_copy(src_hbm.at[sid], src_v)
    pltpu.sync_copy(dst_hbm.at[sid], dst_v)
    plsc.subcore_barrier()

    y_half = y_hbm.at[cid]

    for k in range(NBUF - 1):
        pltpu.async_copy(y_half.at[src_v.at[k]], bufs[k], sems[k])

    @pl.loop(0, NCHUNK_S // NBUF)
    def _(i):
        g = NBUF * i
        for k in range(NBUF):
            buf, sem = bufs[k], sems[k]
            nbuf = bufs[(k + NBUF - 1) % NBUF]
            nsem = sems[(k + NBUF - 1) % NBUF]
            pltpu.make_async_copy(y_half.at[src_v.at[g + k]], buf,
                                  sem).wait()
            if k == 0:
                pltpu.async_copy(y_half.at[src_v.at[g + NBUF - 1]], nbuf,
                                 nsem)
            else:
                @pl.when(g + k + NBUF - 1 < NCHUNK_S)
                def _():
                    pltpu.async_copy(y_half.at[src_v.at[g + k + NBUF - 1]],
                                     nbuf, nsem)
            pltpu.sync_copy(buf, acc.at[dst_v.at[g + k]], add=True)

    plsc.subcore_barrier()
    pltpu.sync_copy(acc.at[pl.ds(sid * RPS, RPS)], out_hbm.at[cid].at[sid])


RB = 2000


def _prep_body(x_ref, w_ref, d0_ref, d1_ref, y_ref, dis_ref):
    dis = lax.rsqrt(1.0 + d0_ref[...] + d1_ref[...])
    dis_ref[...] = dis
    xw = jnp.dot(x_ref[...], w_ref[...], preferred_element_type=jnp.float32)
    y_ref[0, ...] = dis * xw[:, :H]
    y_ref[1, ...] = dis * xw[:, H:]


def _prep(x, w, d0, d1):
    return pl.pallas_call(
        _prep_body,
        grid=(N // RB,),
        in_specs=[pl.BlockSpec((RB, D), lambda i: (i, 0)),
                  pl.BlockSpec((D, D), lambda i: (0, 0)),
                  pl.BlockSpec((RB, 1), lambda i: (i, 0)),
                  pl.BlockSpec((RB, 1), lambda i: (i, 0))],
        out_specs=[pl.BlockSpec((NC, RB, H), lambda i: (0, i, 0)),
                   pl.BlockSpec((RB, 1), lambda i: (i, 0))],
        out_shape=[jax.ShapeDtypeStruct((NC, N, H), jnp.float32),
                   jax.ShapeDtypeStruct((N, 1), jnp.float32)],
    )(x, w, d0, d1)


def _mid_body(s_ref, y_ref, dis_ref, b_ref, w_ref, y2_ref):
    dis = dis_ref[...]
    h0 = jnp.maximum(dis * (s_ref[0, ...] + y_ref[0, ...]) + b_ref[:, :H], 0.0)
    h1 = jnp.maximum(dis * (s_ref[1, ...] + y_ref[1, ...]) + b_ref[:, H:], 0.0)
    xw = (jnp.dot(h0, w_ref[:H, :], preferred_element_type=jnp.float32)
          + jnp.dot(h1, w_ref[H:, :], preferred_element_type=jnp.float32))
    y2_ref[0, ...] = dis * xw[:, :H]
    y2_ref[1, ...] = dis * xw[:, H:]


def _mid(s, y, dis, b, w):
    return pl.pallas_call(
        _mid_body,
        grid=(N // RB,),
        in_specs=[pl.BlockSpec((NC, RB, H), lambda i: (0, i, 0)),
                  pl.BlockSpec((NC, RB, H), lambda i: (0, i, 0)),
                  pl.BlockSpec((RB, 1), lambda i: (i, 0)),
                  pl.BlockSpec((1, D), lambda i: (0, 0)),
                  pl.BlockSpec((D, D), lambda i: (0, 0))],
        out_specs=pl.BlockSpec((NC, RB, H), lambda i: (0, i, 0)),
        out_shape=jax.ShapeDtypeStruct((NC, N, H), jnp.float32),
    )(s, y, dis, b, w)


def _fin_body(s_ref, y_ref, dis_ref, b_ref, o_ref):
    dis = dis_ref[...]
    o_ref[:, :H] = dis * (s_ref[0, ...] + y_ref[0, ...]) + b_ref[:, :H]
    o_ref[:, H:] = dis * (s_ref[1, ...] + y_ref[1, ...]) + b_ref[:, H:]


def _fin(s, y, dis, b):
    return pl.pallas_call(
        _fin_body,
        grid=(N // RB,),
        in_specs=[pl.BlockSpec((NC, RB, H), lambda i: (0, i, 0)),
                  pl.BlockSpec((NC, RB, H), lambda i: (0, i, 0)),
                  pl.BlockSpec((RB, 1), lambda i: (i, 0)),
                  pl.BlockSpec((1, D), lambda i: (0, 0))],
        out_specs=pl.BlockSpec((RB, D), lambda i: (i, 0)),
        out_shape=jax.ShapeDtypeStruct((N, D), jnp.float32),
    )(s, y, dis, b)


def kernel(x, edge_idx, W1, b1, W2, b2):
    src3 = edge_idx[0].reshape(NS, NCHUNK_S, CH)
    dst3 = edge_idx[1].reshape(NS, NCHUNK_S, CH)
    dst3d = edge_idx[1].reshape(NW, NCHUNK_D, CH)

    deg_parts = _deg_kernel(dst3d)
    deg_parts = deg_parts.reshape(NC, N, DEGW)
    d0 = deg_parts[0, :, 0:1]
    d1 = deg_parts[1, :, 0:1]

    y1, dis = _prep(x, W1, d0, d1)

    s1 = _seg_kernel(y1, src3, dst3).reshape(NC, N, H)
    y2 = _mid(s1, y1, dis, b1.reshape(1, D), W2)

    s2 = _seg_kernel(y2, src3, dst3).reshape(NC, N, H)
    return _fin(s2, y2, dis, b2.reshape(1, D))
